# baseline (speedup 1.0000x reference)
import functools

import jax
import jax.numpy as jnp
from jax.experimental import pallas as pl
from jax.experimental.pallas import tpu as pltpu

N_PAD = 50176
ROW_BLK = 512


def _mlp_body(h_ref, l1w_ref, l1b_ref, l2w_ref, l2b_ref, out_ref):
    h = h_ref[...]
    a = h @ l1w_ref[...] + l1b_ref[...]
    z = jnp.where(a > 0.0, a, jnp.exp(jnp.minimum(a, 0.0)) - 1.0)
    o = z @ l2w_ref[...] + l2b_ref[...]
    m = jnp.max(o, axis=1, keepdims=True)
    lse = jnp.log(jnp.sum(jnp.exp(o - m), axis=1, keepdims=True)) + m
    out_ref[...] = o - lse


def _mlp_tail(h, l1w, l1b, l2w, l2b):
    n = h.shape[0]
    h = jnp.pad(h, ((0, N_PAD - n), (0, 0)))
    out = pl.pallas_call(
        _mlp_body,
        grid=(N_PAD // ROW_BLK,),
        in_specs=[
            pl.BlockSpec((ROW_BLK, 64), lambda i: (i, 0)),
            pl.BlockSpec((64, 128), lambda i: (0, 0)),
            pl.BlockSpec((128,), lambda i: (0,)),
            pl.BlockSpec((128, 10), lambda i: (0, 0)),
            pl.BlockSpec((10,), lambda i: (0,)),
        ],
        out_specs=pl.BlockSpec((ROW_BLK, 10), lambda i: (i, 0)),
        out_shape=jax.ShapeDtypeStruct((N_PAD, 10), jnp.float32),
    )(h, l1w, l1b, l2w, l2b)
    return out[:n]


def _spline_agg(h, src, dst, edge_attr, W, K=5):
    n = h.shape[0]
    h_src = h[src]
    u = jnp.clip(edge_attr[:, 0], 0.0, 1.0) * (K - 1)
    bot = jnp.floor(u)
    frac = u - bot
    bot_i = bot.astype(jnp.int32)
    top_i = jnp.minimum(bot_i + 1, K - 1)
    acc = jnp.zeros((n, W.shape[2]), dtype=h.dtype)
    for k in range(K):
        w_k = (1.0 - frac) * (bot_i == k) + frac * (top_i == k)
        gathered = jax.ops.segment_sum(w_k[:, None] * h_src, dst, num_segments=n)
        acc = acc + gathered @ W[k]
    return acc


def kernel(x, edge_index, edge_attr, W1, root1, b1, W2, root2, b2, l1w, l1b, l2w, l2b):
    src = edge_index[0].astype(jnp.int32)
    dst = edge_index[1].astype(jnp.int32)
    n = x.shape[0]
    deg = jax.ops.segment_sum(jnp.ones(src.shape, jnp.float32), dst, num_segments=n)
    inv_deg = 1.0 / jnp.maximum(deg, 1.0)

    a1 = _spline_agg(x, src, dst, edge_attr, W1)
    h1 = jax.nn.elu(a1 * inv_deg[:, None] + x @ root1 + b1)
    a2 = _spline_agg(h1, src, dst, edge_attr, W2)
    h2 = jax.nn.elu(a2 * inv_deg[:, None] + h1 @ root2 + b2)
    return _mlp_tail(h2, l1w, l1b, l2w, l2b)


# speedup vs baseline: 1.1921x; 1.1921x over previous
import jax
import jax.numpy as jnp
from jax import lax
from jax.experimental import pallas as pl
from jax.experimental.pallas import tpu as pltpu
from jax.experimental.pallas import tpu_sc as plsc

N = 50000
E = 800000
EP = 819200
L1ACC = 401408
L1TILE = 25088
R2 = 3096
DUMP2 = R2 * 5
R2ROWS = 15488
R2TILE = 968
BLK = 1000

_mesh = plsc.VectorSubcoreMesh(core_axis_name="c", subcore_axis_name="s")


def _l1_body(x_hbm, src_hbm, dst_hbm, attr_hbm, z_hbm, out_hbm,
             srcb, dstb, attrb, xloc, vals, idxs, acc, sem):
    c = lax.axis_index("c")
    s = lax.axis_index("s")
    wid = s * 2 + c
    pltpu.sync_copy(x_hbm, xloc)
    pltpu.sync_copy(z_hbm.at[pl.ds(s * L1TILE, L1TILE)],
                    acc.at[pl.ds(s * L1TILE, L1TILE)])
    plsc.subcore_barrier()
    ones = jnp.full((16,), 1.0, jnp.float32)
    for g in range(16):
        vals[pl.ds(512 + g * 16, 16)] = ones

    base_e = wid * (EP // 32)

    def batch(b, carry):
        off = base_e + b * 256
        pltpu.sync_copy(src_hbm.at[pl.ds(off, 256)], srcb)
        pltpu.sync_copy(dst_hbm.at[pl.ds(off, 256)], dstb)
        pltpu.sync_copy(attr_hbm.at[pl.ds(off, 256)], attrb)
        for g in range(16):
            sl = pl.ds(g * 16, 16)
            a = attrb[sl]
            u = jnp.minimum(jnp.maximum(a, 0.0), 1.0) * 4.0
            bi = u.astype(jnp.int32)
            f = u - bi.astype(jnp.float32)
            top = jnp.minimum(bi + 1, 4)
            xs = plsc.load_gather(xloc, [srcb[sl]])
            g1 = f * xs
            d8 = dstb[sl] * 8
            vals[sl] = xs - g1
            vals[pl.ds(256 + g * 16, 16)] = g1
            idxs[sl] = d8 + bi
            idxs[pl.ds(256 + g * 16, 16)] = d8 + top
            idxs[pl.ds(512 + g * 16, 16)] = d8 + 5
        pltpu.sync_copy(vals, acc.at[idxs], add=True)
        return carry

    lax.fori_loop(0, EP // 32 // 256, batch, 0)
    plsc.subcore_barrier()
    pltpu.sync_copy(acc.at[pl.ds(s * L1TILE, L1TILE)],
                    out_hbm.at[pl.ds(c * L1ACC + s * L1TILE, L1TILE)])


_l1_call = pl.kernel(
    _l1_body,
    out_type=[jax.ShapeDtypeStruct((2 * L1ACC,), jnp.float32)],
    mesh=_mesh,
    compiler_params=pltpu.CompilerParams(use_tc_tiling_on_sc=False, needs_layout_passes=False),
    scratch_types=[
        pltpu.VMEM((256,), jnp.int32),
        pltpu.VMEM((256,), jnp.int32),
        pltpu.VMEM((256,), jnp.float32),
        pltpu.VMEM((N,), jnp.float32),
        pltpu.VMEM((768,), jnp.float32),
        pltpu.VMEM((768,), jnp.int32),
        pltpu.VMEM_SHARED((L1ACC,), jnp.float32),
        pltpu.SemaphoreType.DMA,
    ],
)


def _l2_body(h1_hbm, src_hbm, dst_hbm, attr_hbm, z_hbm, out_hbm,
             srcb, dstb, attrb, hrows, gb, db, ib, it, fbuf, acc, sem):
    c = lax.axis_index("c")
    s = lax.axis_index("s")

    def one_pass(p, carry0):
        r = p * 2 + c
        base = r * R2
        pltpu.sync_copy(z_hbm.at[pl.ds(s * R2TILE, R2TILE)],
                        acc.at[pl.ds(s * R2TILE, R2TILE)])
        plsc.subcore_barrier()

        def batch(b, carry):
            off = s * (EP // 16) + b * 256
            pltpu.sync_copy(src_hbm.at[pl.ds(off, 256)], srcb)
            pltpu.sync_copy(dst_hbm.at[pl.ds(off, 256)], dstb)
            pltpu.sync_copy(attr_hbm.at[pl.ds(off, 256)], attrb)
            pltpu.async_copy(h1_hbm.at[srcb], hrows, sem).wait()

            def group(g, carry2):
                sl = pl.ds(g * 16, 16)
                a = attrb[sl]
                u = jnp.minimum(jnp.maximum(a, 0.0), 1.0) * 4.0
                bi = u.astype(jnp.int32)
                f = u - bi.astype(jnp.float32)
                top = jnp.minimum(bi + 1, 4)
                loc = dstb[sl] - base
                inr = (loc >= 0) & (loc < R2)
                r5 = loc * 5
                ib[sl] = jnp.where(inr, r5 + bi, DUMP2)
                it[sl] = jnp.where(inr, r5 + top, DUMP2)
                fbuf[sl] = f
                return carry2

            lax.fori_loop(0, 16, group, 0)

            def epack(g, carry3):
                fs_vec = fbuf[pl.ds(g * 16, 16)]
                for l in range(16):
                    e = g * 16 + l
                    fs = fs_vec[l]
                    h0 = hrows[e, pl.ds(0, 16)]
                    h1v = hrows[e, pl.ds(16, 16)]
                    g0 = fs * h0
                    g1 = fs * h1v
                    gb[e, pl.ds(0, 16)] = g0
                    gb[e, pl.ds(16, 16)] = g1
                    db[e, pl.ds(0, 16)] = h0 - g0
                    db[e, pl.ds(16, 16)] = h1v - g1
                return carry3

            lax.fori_loop(0, 16, epack, 0)
            pltpu.sync_copy(db, acc.at[ib], add=True)
            pltpu.sync_copy(gb, acc.at[it], add=True)
            return carry

        lax.fori_loop(0, EP // 16 // 256, batch, 0)
        plsc.subcore_barrier()
        pltpu.sync_copy(acc.at[pl.ds(s * R2TILE, R2TILE)],
                        out_hbm.at[pl.ds(r * R2ROWS + s * R2TILE, R2TILE)])
        plsc.subcore_barrier()
        return carry0

    lax.fori_loop(0, 9, one_pass, 0)


_l2_call = pl.kernel(
    _l2_body,
    out_type=[jax.ShapeDtypeStruct((18 * R2ROWS, 32), jnp.float32)],
    mesh=_mesh,
    compiler_params=pltpu.CompilerParams(use_tc_tiling_on_sc=False, needs_layout_passes=False),
    scratch_types=[
        pltpu.VMEM((256,), jnp.int32),
        pltpu.VMEM((256,), jnp.int32),
        pltpu.VMEM((256,), jnp.float32),
        pltpu.VMEM((256, 32), jnp.float32),
        pltpu.VMEM((256, 32), jnp.float32),
        pltpu.VMEM((256, 32), jnp.float32),
        pltpu.VMEM((256,), jnp.int32),
        pltpu.VMEM((256,), jnp.int32),
        pltpu.VMEM((256,), jnp.float32),
        pltpu.VMEM_SHARED((R2ROWS, 32), jnp.float32),
        pltpu.SemaphoreType.DMA,
    ],
)


def _tca_body(s1_ref, x_ref, w1f_ref, root1_ref, b1_ref, h1_ref, inv_ref):
    sm = s1_ref[0] + s1_ref[1]
    inv = 1.0 / jnp.maximum(sm[:, 5:6], 1.0)
    acc = sm @ w1f_ref[...]
    z = acc * inv + x_ref[...] * root1_ref[...] + b1_ref[...]
    h1_ref[...] = jnp.where(z > 0.0, z, jnp.exp(jnp.minimum(z, 0.0)) - 1.0)
    inv_ref[...] = jnp.broadcast_to(inv, (BLK, 8))


def _tca(s1r, x, w1f, root1, b1):
    return pl.pallas_call(
        _tca_body,
        grid=(N // BLK,),
        in_specs=[
            pl.BlockSpec((2, BLK, 8), lambda i: (0, i, 0)),
            pl.BlockSpec((BLK, 1), lambda i: (i, 0)),
            pl.BlockSpec((8, 32), lambda i: (0, 0)),
            pl.BlockSpec((1, 32), lambda i: (0, 0)),
            pl.BlockSpec((1, 32), lambda i: (0, 0)),
        ],
        out_specs=[
            pl.BlockSpec((BLK, 32), lambda i: (i, 0)),
            pl.BlockSpec((BLK, 8), lambda i: (i, 0)),
        ],
        out_shape=[
            jax.ShapeDtypeStruct((N, 32), jnp.float32),
            jax.ShapeDtypeStruct((N, 8), jnp.float32),
        ],
    )(s1r, x, w1f, root1, b1)


def _tcb_body(s2_ref, h1_ref, inv_ref, w2f_ref, root2_ref, b2_ref,
              l1w_ref, l1b_ref, l2w_ref, l2b_ref, out_ref):
    inv = inv_ref[:, 0:1]
    z = s2_ref[...] @ w2f_ref[...] * inv + h1_ref[...] @ root2_ref[...] + b2_ref[...]
    h2 = jnp.where(z > 0.0, z, jnp.exp(jnp.minimum(z, 0.0)) - 1.0)
    a = h2 @ l1w_ref[...] + l1b_ref[...]
    t = jnp.where(a > 0.0, a, jnp.exp(jnp.minimum(a, 0.0)) - 1.0)
    o = t @ l2w_ref[...] + l2b_ref[...]
    mask = lax.broadcasted_iota(jnp.int32, o.shape, 1) < 10
    o = jnp.where(mask, o, -1e30)
    m = jnp.max(o, axis=1, keepdims=True)
    lse = jnp.log(jnp.sum(jnp.exp(o - m), axis=1, keepdims=True)) + m
    out_ref[...] = o - lse


def _tcb(s2, h1, inv8, w2f, root2, b2, l1w, l1b, l2wp, l2bp):
    return pl.pallas_call(
        _tcb_body,
        grid=(N // BLK,),
        in_specs=[
            pl.BlockSpec((BLK, 160), lambda i: (i, 0)),
            pl.BlockSpec((BLK, 32), lambda i: (i, 0)),
            pl.BlockSpec((BLK, 8), lambda i: (i, 0)),
            pl.BlockSpec((160, 64), lambda i: (0, 0)),
            pl.BlockSpec((32, 64), lambda i: (0, 0)),
            pl.BlockSpec((1, 64), lambda i: (0, 0)),
            pl.BlockSpec((64, 128), lambda i: (0, 0)),
            pl.BlockSpec((1, 128), lambda i: (0, 0)),
            pl.BlockSpec((128, 128), lambda i: (0, 0)),
            pl.BlockSpec((1, 128), lambda i: (0, 0)),
        ],
        out_specs=pl.BlockSpec((BLK, 128), lambda i: (i, 0)),
        out_shape=jax.ShapeDtypeStruct((N, 128), jnp.float32),
    )(s2, h1, inv8, w2f, root2, b2, l1w, l1b, l2wp, l2bp)


def kernel(x, edge_index, edge_attr, W1, root1, b1, W2, root2, b2, l1w, l1b, l2w, l2b):
    src = edge_index[0].astype(jnp.int32)
    dst = edge_index[1].astype(jnp.int32)
    attr = edge_attr[:, 0]
    pad = EP - E
    src_p = jnp.concatenate([src, jnp.zeros((pad,), jnp.int32)])
    dst_p = jnp.concatenate([dst, jnp.full((pad,), N, jnp.int32)])
    attr_p = jnp.concatenate([attr, jnp.zeros((pad,), jnp.float32)])

    x_flat = x[:, 0]
    z1 = jnp.zeros((L1ACC,), jnp.float32)
    (s1_pair,) = _l1_call(x_flat, src_p, dst_p, attr_p, z1)
    s1r = s1_pair.reshape(2, L1ACC)[:, : N * 8].reshape(2, N, 8)

    w1f = jnp.pad(W1[:, 0, :], ((0, 3), (0, 0)))
    h1, inv8 = _tca(s1r, x, w1f, root1, b1.reshape(1, 32))

    z2 = jnp.zeros((R2ROWS, 32), jnp.float32)
    (s2_parts,) = _l2_call(h1, src_p, dst_p, attr_p, z2)
    s2_parts = s2_parts.reshape(18, R2ROWS, 32)
    chunks = []
    for r in range(17):
        n_r = min(R2, N - r * R2)
        chunks.append(s2_parts[r, : n_r * 5].reshape(n_r, 160))
    s2 = jnp.concatenate(chunks)

    w2f = W2.reshape(160, 64)
    l2wp = jnp.pad(l2w, ((0, 0), (0, 118)))
    l2bp = jnp.pad(l2b, (0, 118)).reshape(1, 128)
    out = _tcb(s2, h1, inv8, w2f, root2, b2.reshape(1, 64),
               l1w, l1b.reshape(1, 128), l2wp, l2bp)
    return out[:, :10]


# speedup vs baseline: 4.3242x; 3.6275x over previous
import jax
import jax.numpy as jnp
from jax import lax
from jax.experimental import pallas as pl
from jax.experimental.pallas import tpu as pltpu
from jax.experimental.pallas import tpu_sc as plsc

N = 50000
E = 800000
EP = 819200
L1ACC = 401408
L1TILE = 25088
R2 = 3096
DUMP2 = R2 * 5
R2ROWS = 15488
R2TILE = 968
BLK = 1000

_mesh = plsc.VectorSubcoreMesh(core_axis_name="c", subcore_axis_name="s")
_sc_params = pltpu.CompilerParams(use_tc_tiling_on_sc=False,
                                  needs_layout_passes=False)


def _l1_body(x_hbm, rec_hbm, z_hbm, out_hbm, rec, xloc, vals, idxs, acc, sem):
    c = lax.axis_index("c")
    s = lax.axis_index("s")
    wid = s * 2 + c
    iota16 = lax.broadcasted_iota(jnp.int32, (16,), 0)
    pltpu.sync_copy(x_hbm, xloc)
    pltpu.sync_copy(z_hbm.at[pl.ds(s * L1TILE, L1TILE)],
                    acc.at[pl.ds(s * L1TILE, L1TILE)])
    plsc.subcore_barrier()
    ones = jnp.full((16,), 1.0, jnp.float32)
    for g in range(16):
        vals[pl.ds(512 + g * 16, 16)] = ones

    base_e = wid * (EP // 32)

    def batch(b, carry):
        off = base_e + b * 256
        pltpu.sync_copy(rec_hbm.at[pl.ds(off * 4, 1024)], rec)
        for g in range(16):
            sl = pl.ds(g * 16, 16)
            li = (iota16 + g * 16) * 4
            sv = plsc.load_gather(rec, [li])
            dv = plsc.load_gather(rec, [li + 1])
            a = plsc.bitcast(plsc.load_gather(rec, [li + 2]), jnp.float32)
            u = jnp.minimum(jnp.maximum(a, 0.0), 1.0) * 4.0
            bi = u.astype(jnp.int32)
            f = u - bi.astype(jnp.float32)
            top = jnp.minimum(bi + 1, 4)
            xs = plsc.load_gather(xloc, [sv])
            g1 = f * xs
            d8 = dv * 8
            vals[sl] = xs - g1
            vals[pl.ds(256 + g * 16, 16)] = g1
            idxs[sl] = d8 + bi
            idxs[pl.ds(256 + g * 16, 16)] = d8 + top
            idxs[pl.ds(512 + g * 16, 16)] = d8 + 5
        pltpu.sync_copy(vals, acc.at[idxs], add=True)
        return carry

    lax.fori_loop(0, EP // 32 // 256, batch, 0)
    plsc.subcore_barrier()
    pltpu.sync_copy(acc.at[pl.ds(s * L1TILE, L1TILE)],
                    out_hbm.at[pl.ds(c * L1ACC + s * L1TILE, L1TILE)])


_l1_call = pl.kernel(
    _l1_body,
    out_type=[jax.ShapeDtypeStruct((2 * L1ACC,), jnp.float32)],
    mesh=_mesh,
    compiler_params=_sc_params,
    scratch_types=[
        pltpu.VMEM((1024,), jnp.int32),
        pltpu.VMEM((N,), jnp.float32),
        pltpu.VMEM((768,), jnp.float32),
        pltpu.VMEM((768,), jnp.int32),
        pltpu.VMEM_SHARED((L1ACC,), jnp.float32),
        pltpu.SemaphoreType.DMA,
    ],
)


def _l2_body(h1_hbm, rec_hbm, z_hbm, out_hbm,
             rec, srcb, ib, it, fb, hrows, gb, db, csrc, crb, crt, cf, acc, sem):
    c = lax.axis_index("c")
    s = lax.axis_index("s")
    iota16 = lax.broadcasted_iota(jnp.int32, (16,), 0)

    def do_flush():
        pltpu.async_copy(h1_hbm.at[srcb], hrows, sem).wait()

        def epack(g, carry3):
            fs_vec = fb[pl.ds(g * 16, 16)]
            for l in range(16):
                e = g * 16 + l
                fs = fs_vec[l]
                h0 = hrows[e, pl.ds(0, 16)]
                h1v = hrows[e, pl.ds(16, 16)]
                g0 = fs * h0
                g1 = fs * h1v
                gb[e, pl.ds(0, 16)] = g0
                gb[e, pl.ds(16, 16)] = g1
                db[e, pl.ds(0, 16)] = h0 - g0
                db[e, pl.ds(16, 16)] = h1v - g1
            return carry3

        lax.fori_loop(0, 16, epack, 0)
        pltpu.sync_copy(db, acc.at[ib], add=True)
        pltpu.sync_copy(gb, acc.at[it], add=True)

    def flush_full():
        def mv(i, carry):
            sl = pl.ds(i * 16, 16)
            srcb[sl] = csrc[sl]
            ib[sl] = crb[sl]
            it[sl] = crt[sl]
            fb[sl] = cf[sl]
            return carry

        lax.fori_loop(0, 16, mv, 0)
        do_flush()

    def flush_masked(off, cur):
        def mv(i, carry):
            sl = pl.ds(i * 16, 16)
            sl2 = pl.ds(off + i * 16, 16)
            valid = (iota16 + (off + i * 16)) < cur
            srcb[sl] = jnp.where(valid, csrc[sl2], 0)
            ib[sl] = jnp.where(valid, crb[sl2], DUMP2)
            it[sl] = jnp.where(valid, crt[sl2], DUMP2)
            fb[sl] = cf[sl2]
            return carry

        lax.fori_loop(0, 16, mv, 0)
        do_flush()

    def one_pass(p, carry0):
        r = p * 2 + c
        base = r * R2
        pltpu.sync_copy(z_hbm.at[pl.ds(s * R2TILE, R2TILE)],
                        acc.at[pl.ds(s * R2TILE, R2TILE)])
        plsc.subcore_barrier()

        def batch(b, cur):
            off = s * (EP // 16) + b * 256
            pltpu.sync_copy(rec_hbm.at[pl.ds(off * 4, 1024)], rec)

            def group(g, cur2):
                li = (iota16 + g * 16) * 4
                sv = plsc.load_gather(rec, [li])
                dv = plsc.load_gather(rec, [li + 1])
                a = plsc.bitcast(plsc.load_gather(rec, [li + 2]), jnp.float32)
                u = jnp.minimum(jnp.maximum(a, 0.0), 1.0) * 4.0
                bi = u.astype(jnp.int32)
                f = u - bi.astype(jnp.float32)
                top = jnp.minimum(bi + 1, 4)
                loc = dv - base
                inr = (loc >= 0) & (loc < R2)
                r5 = loc * 5
                csl = pl.ds(cur2, 16)
                plsc.store_compressed(csrc.at[csl], sv, mask=inr)
                plsc.store_compressed(crb.at[csl], r5 + bi, mask=inr)
                plsc.store_compressed(crt.at[csl], r5 + top, mask=inr)
                plsc.store_compressed(cf.at[csl], f, mask=inr)
                cnt = plsc.all_reduce_population_count(inr)[0]
                return cur2 + cnt

            cur = lax.fori_loop(0, 16, group, cur)

            def spill(cv):
                flush_full()

                def sh(i, carry):
                    sl = pl.ds(i * 16, 16)
                    sl2 = pl.ds(256 + i * 16, 16)
                    csrc[sl] = csrc[sl2]
                    crb[sl] = crb[sl2]
                    crt[sl] = crt[sl2]
                    cf[sl] = cf[sl2]
                    return carry

                lax.fori_loop(0, 16, sh, 0)
                return cv - 256

            return lax.cond(cur >= 256, spill, lambda cv: cv, cur)

        cur = lax.fori_loop(0, EP // 16 // 256, batch, 0)
        flush_masked(0, cur)
        flush_masked(256, cur)
        plsc.subcore_barrier()
        pltpu.sync_copy(acc.at[pl.ds(s * R2TILE, R2TILE)],
                        out_hbm.at[pl.ds(r * R2ROWS + s * R2TILE, R2TILE)])
        plsc.subcore_barrier()
        return carry0

    lax.fori_loop(0, 9, one_pass, 0)


_l2_call = pl.kernel(
    _l2_body,
    out_type=[jax.ShapeDtypeStruct((18 * R2ROWS, 32), jnp.float32)],
    mesh=_mesh,
    compiler_params=_sc_params,
    scratch_types=[
        pltpu.VMEM((1024,), jnp.int32),
        pltpu.VMEM((256,), jnp.int32),
        pltpu.VMEM((256,), jnp.int32),
        pltpu.VMEM((256,), jnp.int32),
        pltpu.VMEM((256,), jnp.float32),
        pltpu.VMEM((256, 32), jnp.float32),
        pltpu.VMEM((256, 32), jnp.float32),
        pltpu.VMEM((256, 32), jnp.float32),
        pltpu.VMEM((512,), jnp.int32),
        pltpu.VMEM((512,), jnp.int32),
        pltpu.VMEM((512,), jnp.int32),
        pltpu.VMEM((512,), jnp.float32),
        pltpu.VMEM_SHARED((R2ROWS, 32), jnp.float32),
        pltpu.SemaphoreType.DMA,
    ],
)


def _tca_body(s1_ref, x_ref, w1f_ref, root1_ref, b1_ref, h1_ref, inv_ref):
    sm = s1_ref[0] + s1_ref[1]
    inv = 1.0 / jnp.maximum(sm[:, 5:6], 1.0)
    acc = sm @ w1f_ref[...]
    z = acc * inv + x_ref[...] * root1_ref[...] + b1_ref[...]
    h1_ref[...] = jnp.where(z > 0.0, z, jnp.exp(jnp.minimum(z, 0.0)) - 1.0)
    inv_ref[...] = jnp.broadcast_to(inv, (BLK, 8))


def _tca(s1r, x, w1f, root1, b1):
    return pl.pallas_call(
        _tca_body,
        grid=(N // BLK,),
        in_specs=[
            pl.BlockSpec((2, BLK, 8), lambda i: (0, i, 0)),
            pl.BlockSpec((BLK, 1), lambda i: (i, 0)),
            pl.BlockSpec((8, 32), lambda i: (0, 0)),
            pl.BlockSpec((1, 32), lambda i: (0, 0)),
            pl.BlockSpec((1, 32), lambda i: (0, 0)),
        ],
        out_specs=[
            pl.BlockSpec((BLK, 32), lambda i: (i, 0)),
            pl.BlockSpec((BLK, 8), lambda i: (i, 0)),
        ],
        out_shape=[
            jax.ShapeDtypeStruct((N, 32), jnp.float32),
            jax.ShapeDtypeStruct((N, 8), jnp.float32),
        ],
    )(s1r, x, w1f, root1, b1)


def _tcb_body(s2_ref, h1_ref, inv_ref, w2f_ref, root2_ref, b2_ref,
              l1w_ref, l1b_ref, l2w_ref, l2b_ref, out_ref):
    inv = inv_ref[:, 0:1]
    z = s2_ref[...] @ w2f_ref[...] * inv + h1_ref[...] @ root2_ref[...] + b2_ref[...]
    h2 = jnp.where(z > 0.0, z, jnp.exp(jnp.minimum(z, 0.0)) - 1.0)
    a = h2 @ l1w_ref[...] + l1b_ref[...]
    t = jnp.where(a > 0.0, a, jnp.exp(jnp.minimum(a, 0.0)) - 1.0)
    o = t @ l2w_ref[...] + l2b_ref[...]
    mask = lax.broadcasted_iota(jnp.int32, o.shape, 1) < 10
    o = jnp.where(mask, o, -1e30)
    m = jnp.max(o, axis=1, keepdims=True)
    lse = jnp.log(jnp.sum(jnp.exp(o - m), axis=1, keepdims=True)) + m
    out_ref[...] = o - lse


def _tcb(s2, h1, inv8, w2f, root2, b2, l1w, l1b, l2wp, l2bp):
    return pl.pallas_call(
        _tcb_body,
        grid=(N // BLK,),
        in_specs=[
            pl.BlockSpec((BLK, 160), lambda i: (i, 0)),
            pl.BlockSpec((BLK, 32), lambda i: (i, 0)),
            pl.BlockSpec((BLK, 8), lambda i: (i, 0)),
            pl.BlockSpec((160, 64), lambda i: (0, 0)),
            pl.BlockSpec((32, 64), lambda i: (0, 0)),
            pl.BlockSpec((1, 64), lambda i: (0, 0)),
            pl.BlockSpec((64, 128), lambda i: (0, 0)),
            pl.BlockSpec((1, 128), lambda i: (0, 0)),
            pl.BlockSpec((128, 128), lambda i: (0, 0)),
            pl.BlockSpec((1, 128), lambda i: (0, 0)),
        ],
        out_specs=pl.BlockSpec((BLK, 128), lambda i: (i, 0)),
        out_shape=jax.ShapeDtypeStruct((N, 128), jnp.float32),
    )(s2, h1, inv8, w2f, root2, b2, l1w, l1b, l2wp, l2bp)


def kernel(x, edge_index, edge_attr, W1, root1, b1, W2, root2, b2, l1w, l1b, l2w, l2b):
    src = edge_index[0].astype(jnp.int32)
    dst = edge_index[1].astype(jnp.int32)
    attr = edge_attr[:, 0]
    pad = EP - E
    src_p = jnp.concatenate([src, jnp.zeros((pad,), jnp.int32)])
    dst_p = jnp.concatenate([dst, jnp.full((pad,), N, jnp.int32)])
    attr_p = jnp.concatenate([attr, jnp.zeros((pad,), jnp.float32)])
    attr_bits = lax.bitcast_convert_type(attr_p, jnp.int32)
    recs = jnp.stack([src_p, dst_p, attr_bits, jnp.zeros_like(src_p)],
                     axis=1).reshape(-1)

    x_flat = x[:, 0]
    z1 = jnp.zeros((L1ACC,), jnp.float32)
    (s1_pair,) = _l1_call(x_flat, recs, z1)
    s1r = s1_pair.reshape(2, L1ACC)[:, : N * 8].reshape(2, N, 8)

    w1f = jnp.pad(W1[:, 0, :], ((0, 3), (0, 0)))
    h1, inv8 = _tca(s1r, x, w1f, root1, b1.reshape(1, 32))

    z2 = jnp.zeros((R2ROWS, 32), jnp.float32)
    (s2_parts,) = _l2_call(h1, recs, z2)
    s2_parts = s2_parts.reshape(18, R2ROWS, 32)
    chunks = []
    for r in range(17):
        n_r = min(R2, N - r * R2)
        chunks.append(s2_parts[r, : n_r * 5].reshape(n_r, 160))
    s2 = jnp.concatenate(chunks)

    w2f = W2.reshape(160, 64)
    l2wp = jnp.pad(l2w, ((0, 0), (0, 118)))
    l2bp = jnp.pad(l2b, (0, 118)).reshape(1, 128)
    out = _tcb(s2, h1, inv8, w2f, root2, b2.reshape(1, 64),
               l1w, l1b.reshape(1, 128), l2wp, l2bp)
    return out[:, :10]


# speedup vs baseline: 4.6891x; 1.0844x over previous
import jax
import jax.numpy as jnp
from jax import lax
from jax.experimental import pallas as pl
from jax.experimental.pallas import tpu as pltpu
from jax.experimental.pallas import tpu_sc as plsc

N = 50000
E = 800000
EP = 819200
L1ACC = 401408
L1TILE = 25088
R2 = 3096
DUMP2 = R2 * 5
R2ROWS = 15488
R2TILE = 968
BLK = 1000

_mesh = plsc.VectorSubcoreMesh(core_axis_name="c", subcore_axis_name="s")
_sc_params = pltpu.CompilerParams(use_tc_tiling_on_sc=False,
                                  needs_layout_passes=False)


def _l1_body(x_hbm, rec_hbm, z_hbm, out_hbm, rec, xloc, vals, idxs, acc, sem):
    c = lax.axis_index("c")
    s = lax.axis_index("s")
    wid = s * 2 + c
    iota16 = lax.broadcasted_iota(jnp.int32, (16,), 0)
    pltpu.sync_copy(x_hbm, xloc)
    pltpu.sync_copy(z_hbm.at[pl.ds(s * L1TILE, L1TILE)],
                    acc.at[pl.ds(s * L1TILE, L1TILE)])
    plsc.subcore_barrier()
    ones = jnp.full((16,), 1.0, jnp.float32)
    for g in range(16):
        vals[pl.ds(512 + g * 16, 16)] = ones

    base_e = wid * (EP // 32)

    def batch(b, carry):
        off = base_e + b * 256
        pltpu.sync_copy(rec_hbm.at[pl.ds(off * 4, 1024)], rec)
        for g in range(16):
            sl = pl.ds(g * 16, 16)
            li = (iota16 + g * 16) * 4
            sv = plsc.load_gather(rec, [li])
            dv = plsc.load_gather(rec, [li + 1])
            a = plsc.bitcast(plsc.load_gather(rec, [li + 2]), jnp.float32)
            u = jnp.minimum(jnp.maximum(a, 0.0), 1.0) * 4.0
            bi = u.astype(jnp.int32)
            f = u - bi.astype(jnp.float32)
            top = jnp.minimum(bi + 1, 4)
            xs = plsc.load_gather(xloc, [sv])
            g1 = f * xs
            d8 = dv * 8
            vals[sl] = xs - g1
            vals[pl.ds(256 + g * 16, 16)] = g1
            idxs[sl] = d8 + bi
            idxs[pl.ds(256 + g * 16, 16)] = d8 + top
            idxs[pl.ds(512 + g * 16, 16)] = d8 + 5
        pltpu.sync_copy(vals, acc.at[idxs], add=True)
        return carry

    lax.fori_loop(0, EP // 32 // 256, batch, 0)
    plsc.subcore_barrier()
    pltpu.sync_copy(acc.at[pl.ds(s * L1TILE, L1TILE)],
                    out_hbm.at[pl.ds(c * L1ACC + s * L1TILE, L1TILE)])


_l1_call = pl.kernel(
    _l1_body,
    out_type=[jax.ShapeDtypeStruct((2 * L1ACC,), jnp.float32)],
    mesh=_mesh,
    compiler_params=_sc_params,
    scratch_types=[
        pltpu.VMEM((1024,), jnp.int32),
        pltpu.VMEM((N,), jnp.float32),
        pltpu.VMEM((768,), jnp.float32),
        pltpu.VMEM((768,), jnp.int32),
        pltpu.VMEM_SHARED((L1ACC,), jnp.float32),
        pltpu.SemaphoreType.DMA,
    ],
)


def _l2_body(h1_hbm, rec_hbm, z_hbm, out_hbm,
             recA, recB, srcb, ib, it, fb, hrows, gb, db, csrc, crb, crt, cf,
             acc, sem, semA, semB):
    c = lax.axis_index("c")
    s = lax.axis_index("s")
    iota16 = lax.broadcasted_iota(jnp.int32, (16,), 0)

    def do_flush():
        pltpu.async_copy(h1_hbm.at[srcb], hrows, sem).wait()

        def epack(g, carry3):
            fs_vec = fb[pl.ds(g * 16, 16)]
            for l in range(16):
                e = g * 16 + l
                fs = fs_vec[l]
                h0 = hrows[e, pl.ds(0, 16)]
                h1v = hrows[e, pl.ds(16, 16)]
                g0 = fs * h0
                g1 = fs * h1v
                gb[e, pl.ds(0, 16)] = g0
                gb[e, pl.ds(16, 16)] = g1
                db[e, pl.ds(0, 16)] = h0 - g0
                db[e, pl.ds(16, 16)] = h1v - g1
            return carry3

        lax.fori_loop(0, 16, epack, 0)
        pltpu.sync_copy(db, acc.at[ib], add=True)
        pltpu.sync_copy(gb, acc.at[it], add=True)

    def flush_full():
        def mv(i, carry):
            sl = pl.ds(i * 16, 16)
            srcb[sl] = csrc[sl]
            ib[sl] = crb[sl]
            it[sl] = crt[sl]
            fb[sl] = cf[sl]
            return carry

        lax.fori_loop(0, 16, mv, 0)
        do_flush()

    def flush_masked(off, cur):
        def mv(i, carry):
            sl = pl.ds(i * 16, 16)
            sl2 = pl.ds(off + i * 16, 16)
            valid = (iota16 + (off + i * 16)) < cur
            srcb[sl] = jnp.where(valid, csrc[sl2], 0)
            ib[sl] = jnp.where(valid, crb[sl2], DUMP2)
            it[sl] = jnp.where(valid, crt[sl2], DUMP2)
            fb[sl] = cf[sl2]
            return carry

        lax.fori_loop(0, 16, mv, 0)
        do_flush()

    def one_pass(p, carry0):
        r = p * 2 + c
        base = r * R2
        pltpu.sync_copy(z_hbm.at[pl.ds(s * R2TILE, R2TILE)],
                        acc.at[pl.ds(s * R2TILE, R2TILE)])
        plsc.subcore_barrier()

        def process(rec, cur):
            def group(g, cur2):
                li = (iota16 + g * 16) * 4
                sv = plsc.load_gather(rec, [li])
                dv = plsc.load_gather(rec, [li + 1])
                a = plsc.bitcast(plsc.load_gather(rec, [li + 2]), jnp.float32)
                u = jnp.minimum(jnp.maximum(a, 0.0), 1.0) * 4.0
                bi = u.astype(jnp.int32)
                f = u - bi.astype(jnp.float32)
                top = jnp.minimum(bi + 1, 4)
                loc = dv - base
                inr = (loc >= 0) & (loc < R2)
                r5 = loc * 5
                csl = pl.ds(cur2, 16)
                plsc.store_compressed(csrc.at[csl], sv, mask=inr)
                plsc.store_compressed(crb.at[csl], r5 + bi, mask=inr)
                plsc.store_compressed(crt.at[csl], r5 + top, mask=inr)
                plsc.store_compressed(cf.at[csl], f, mask=inr)
                cnt = plsc.all_reduce_population_count(inr)[0]
                return cur2 + cnt

            cur = lax.fori_loop(0, 16, group, cur)

            def spill(cv):
                flush_full()

                def sh(i, carry):
                    sl = pl.ds(i * 16, 16)
                    sl2 = pl.ds(256 + i * 16, 16)
                    csrc[sl] = csrc[sl2]
                    crb[sl] = crb[sl2]
                    crt[sl] = crt[sl2]
                    cf[sl] = cf[sl2]
                    return carry

                lax.fori_loop(0, 16, sh, 0)
                return cv - 256

            return lax.cond(cur >= 256, spill, lambda cv: cv, cur)

        ebase = s * (EP // 16)

        def rec_slice(b):
            return rec_hbm.at[pl.ds((ebase + b * 256) * 4, 1024)]

        pltpu.async_copy(rec_slice(0), recA, semA)

        def batch2(i, cur):
            pltpu.make_async_copy(rec_slice(2 * i), recA, semA).wait()
            pltpu.async_copy(rec_slice(2 * i + 1), recB, semB)
            cur = process(recA, cur)
            pltpu.make_async_copy(rec_slice(2 * i + 1), recB, semB).wait()
            pltpu.async_copy(rec_slice(2 * i + 2), recA, semA)
            return process(recB, cur)

        cur = lax.fori_loop(0, EP // 16 // 512, batch2, 0)
        pltpu.make_async_copy(rec_slice(EP // 16 // 256), recA, semA).wait()
        flush_masked(0, cur)
        flush_masked(256, cur)
        plsc.subcore_barrier()
        pltpu.sync_copy(acc.at[pl.ds(s * R2TILE, R2TILE)],
                        out_hbm.at[pl.ds(r * R2ROWS + s * R2TILE, R2TILE)])
        plsc.subcore_barrier()
        return carry0

    lax.fori_loop(0, 9, one_pass, 0)


_l2_call = pl.kernel(
    _l2_body,
    out_type=[jax.ShapeDtypeStruct((18 * R2ROWS, 32), jnp.float32)],
    mesh=_mesh,
    compiler_params=_sc_params,
    scratch_types=[
        pltpu.VMEM((1024,), jnp.int32),
        pltpu.VMEM((1024,), jnp.int32),
        pltpu.VMEM((256,), jnp.int32),
        pltpu.VMEM((256,), jnp.int32),
        pltpu.VMEM((256,), jnp.int32),
        pltpu.VMEM((256,), jnp.float32),
        pltpu.VMEM((256, 32), jnp.float32),
        pltpu.VMEM((256, 32), jnp.float32),
        pltpu.VMEM((256, 32), jnp.float32),
        pltpu.VMEM((512,), jnp.int32),
        pltpu.VMEM((512,), jnp.int32),
        pltpu.VMEM((512,), jnp.int32),
        pltpu.VMEM((512,), jnp.float32),
        pltpu.VMEM_SHARED((R2ROWS, 32), jnp.float32),
        pltpu.SemaphoreType.DMA,
        pltpu.SemaphoreType.DMA,
        pltpu.SemaphoreType.DMA,
    ],
)


def _tca_body(s1_ref, x_ref, w1f_ref, root1_ref, b1_ref, h1_ref, inv_ref):
    sm = s1_ref[0] + s1_ref[1]
    inv = 1.0 / jnp.maximum(sm[:, 5:6], 1.0)
    acc = sm @ w1f_ref[...]
    z = acc * inv + x_ref[...] * root1_ref[...] + b1_ref[...]
    h1_ref[...] = jnp.where(z > 0.0, z, jnp.exp(jnp.minimum(z, 0.0)) - 1.0)
    inv_ref[...] = jnp.broadcast_to(inv, (BLK, 8))


def _tca(s1r, x, w1f, root1, b1):
    return pl.pallas_call(
        _tca_body,
        grid=(N // BLK,),
        in_specs=[
            pl.BlockSpec((2, BLK, 8), lambda i: (0, i, 0)),
            pl.BlockSpec((BLK, 1), lambda i: (i, 0)),
            pl.BlockSpec((8, 32), lambda i: (0, 0)),
            pl.BlockSpec((1, 32), lambda i: (0, 0)),
            pl.BlockSpec((1, 32), lambda i: (0, 0)),
        ],
        out_specs=[
            pl.BlockSpec((BLK, 32), lambda i: (i, 0)),
            pl.BlockSpec((BLK, 8), lambda i: (i, 0)),
        ],
        out_shape=[
            jax.ShapeDtypeStruct((N, 32), jnp.float32),
            jax.ShapeDtypeStruct((N, 8), jnp.float32),
        ],
    )(s1r, x, w1f, root1, b1)


def _tcb_body(s2_ref, h1_ref, inv_ref, w2f_ref, root2_ref, b2_ref,
              l1w_ref, l1b_ref, l2w_ref, l2b_ref, out_ref):
    inv = inv_ref[:, 0:1]
    z = s2_ref[...] @ w2f_ref[...] * inv + h1_ref[...] @ root2_ref[...] + b2_ref[...]
    h2 = jnp.where(z > 0.0, z, jnp.exp(jnp.minimum(z, 0.0)) - 1.0)
    a = h2 @ l1w_ref[...] + l1b_ref[...]
    t = jnp.where(a > 0.0, a, jnp.exp(jnp.minimum(a, 0.0)) - 1.0)
    o = t @ l2w_ref[...] + l2b_ref[...]
    mask = lax.broadcasted_iota(jnp.int32, o.shape, 1) < 10
    o = jnp.where(mask, o, -1e30)
    m = jnp.max(o, axis=1, keepdims=True)
    lse = jnp.log(jnp.sum(jnp.exp(o - m), axis=1, keepdims=True)) + m
    out_ref[...] = o - lse


def _tcb(s2, h1, inv8, w2f, root2, b2, l1w, l1b, l2wp, l2bp):
    return pl.pallas_call(
        _tcb_body,
        grid=(N // BLK,),
        in_specs=[
            pl.BlockSpec((BLK, 160), lambda i: (i, 0)),
            pl.BlockSpec((BLK, 32), lambda i: (i, 0)),
            pl.BlockSpec((BLK, 8), lambda i: (i, 0)),
            pl.BlockSpec((160, 64), lambda i: (0, 0)),
            pl.BlockSpec((32, 64), lambda i: (0, 0)),
            pl.BlockSpec((1, 64), lambda i: (0, 0)),
            pl.BlockSpec((64, 128), lambda i: (0, 0)),
            pl.BlockSpec((1, 128), lambda i: (0, 0)),
            pl.BlockSpec((128, 128), lambda i: (0, 0)),
            pl.BlockSpec((1, 128), lambda i: (0, 0)),
        ],
        out_specs=pl.BlockSpec((BLK, 128), lambda i: (i, 0)),
        out_shape=jax.ShapeDtypeStruct((N, 128), jnp.float32),
    )(s2, h1, inv8, w2f, root2, b2, l1w, l1b, l2wp, l2bp)


def kernel(x, edge_index, edge_attr, W1, root1, b1, W2, root2, b2, l1w, l1b, l2w, l2b):
    src = edge_index[0].astype(jnp.int32)
    dst = edge_index[1].astype(jnp.int32)
    attr = edge_attr[:, 0]
    pad = EP - E
    src_p = jnp.concatenate([src, jnp.zeros((pad,), jnp.int32)])
    dst_p = jnp.concatenate([dst, jnp.full((pad,), N, jnp.int32)])
    attr_p = jnp.concatenate([attr, jnp.zeros((pad,), jnp.float32)])
    attr_bits = lax.bitcast_convert_type(attr_p, jnp.int32)
    recs = jnp.stack([src_p, dst_p, attr_bits, jnp.zeros_like(src_p)],
                     axis=1).reshape(-1)
    recs = jnp.concatenate([recs, jnp.zeros((1024,), jnp.int32)])

    x_flat = x[:, 0]
    z1 = jnp.zeros((L1ACC,), jnp.float32)
    (s1_pair,) = _l1_call(x_flat, recs, z1)
    s1r = s1_pair.reshape(2, L1ACC)[:, : N * 8].reshape(2, N, 8)

    w1f = jnp.pad(W1[:, 0, :], ((0, 3), (0, 0)))
    h1, inv8 = _tca(s1r, x, w1f, root1, b1.reshape(1, 32))

    z2 = jnp.zeros((R2ROWS, 32), jnp.float32)
    (s2_parts,) = _l2_call(h1, recs, z2)
    s2_parts = s2_parts.reshape(18, R2ROWS, 32)
    chunks = []
    for r in range(17):
        n_r = min(R2, N - r * R2)
        chunks.append(s2_parts[r, : n_r * 5].reshape(n_r, 160))
    s2 = jnp.concatenate(chunks)

    w2f = W2.reshape(160, 64)
    l2wp = jnp.pad(l2w, ((0, 0), (0, 118)))
    l2bp = jnp.pad(l2b, (0, 118)).reshape(1, 128)
    out = _tcb(s2, h1, inv8, w2f, root2, b2.reshape(1, 64),
               l1w, l1b.reshape(1, 128), l2wp, l2bp)
    return out[:, :10]


# speedup vs baseline: 4.6898x; 1.0001x over previous
import jax
import jax.numpy as jnp
from jax import lax
from jax.experimental import pallas as pl
from jax.experimental.pallas import tpu as pltpu
from jax.experimental.pallas import tpu_sc as plsc

N = 50000
E = 800000
EP = 819200
L1ACC = 401408
L1TILE = 25088
R2 = 3096
DUMP2 = 0
R2ROWS = 15488
OUTROWS = 278640
R2TILE = 968
BLK = 1000

_mesh = plsc.VectorSubcoreMesh(core_axis_name="c", subcore_axis_name="s")
_sc_params = pltpu.CompilerParams(use_tc_tiling_on_sc=False,
                                  needs_layout_passes=False)


def _l1_body(x_hbm, rec_hbm, z_hbm, out_hbm, rec, xloc, vals, idxs, acc, sem):
    c = lax.axis_index("c")
    s = lax.axis_index("s")
    wid = s * 2 + c
    iota16 = lax.broadcasted_iota(jnp.int32, (16,), 0)
    pltpu.sync_copy(x_hbm, xloc)
    pltpu.sync_copy(z_hbm.at[pl.ds(s * L1TILE, L1TILE)],
                    acc.at[pl.ds(s * L1TILE, L1TILE)])
    plsc.subcore_barrier()
    ones = jnp.full((16,), 1.0, jnp.float32)
    for g in range(16):
        vals[pl.ds(512 + g * 16, 16)] = ones

    base_e = wid * (EP // 32)

    def batch(b, carry):
        off = base_e + b * 256
        pltpu.sync_copy(rec_hbm.at[pl.ds(off * 4, 1024)], rec)
        for g in range(16):
            sl = pl.ds(g * 16, 16)
            li = (iota16 + g * 16) * 4
            sv = plsc.load_gather(rec, [li])
            dv = plsc.load_gather(rec, [li + 1])
            a = plsc.bitcast(plsc.load_gather(rec, [li + 2]), jnp.float32)
            u = jnp.minimum(jnp.maximum(a, 0.0), 1.0) * 4.0
            bi = u.astype(jnp.int32)
            f = u - bi.astype(jnp.float32)
            top = jnp.minimum(bi + 1, 4)
            xs = plsc.load_gather(xloc, [sv])
            g1 = f * xs
            d8 = dv * 8
            vals[sl] = xs - g1
            vals[pl.ds(256 + g * 16, 16)] = g1
            idxs[sl] = d8 + bi
            idxs[pl.ds(256 + g * 16, 16)] = d8 + top
            idxs[pl.ds(512 + g * 16, 16)] = d8 + 5
        pltpu.sync_copy(vals, acc.at[idxs], add=True)
        return carry

    lax.fori_loop(0, EP // 32 // 256, batch, 0)
    plsc.subcore_barrier()
    pltpu.sync_copy(acc.at[pl.ds(s * L1TILE, L1TILE)],
                    out_hbm.at[pl.ds(c * L1ACC + s * L1TILE, L1TILE)])


_l1_call = pl.kernel(
    _l1_body,
    out_type=[jax.ShapeDtypeStruct((2 * L1ACC,), jnp.float32)],
    mesh=_mesh,
    compiler_params=_sc_params,
    scratch_types=[
        pltpu.VMEM((1024,), jnp.int32),
        pltpu.VMEM((N,), jnp.float32),
        pltpu.VMEM((768,), jnp.float32),
        pltpu.VMEM((768,), jnp.int32),
        pltpu.VMEM_SHARED((L1ACC,), jnp.float32),
        pltpu.SemaphoreType.DMA,
    ],
)


def _l2_body(h1_hbm, rec_hbm, z_hbm, out_hbm,
             recA, recB, srcb, ib, it, fb, hrows, gb, db, csrc, crb, crt, cf,
             acc, sem, semA, semB):
    c = lax.axis_index("c")
    s = lax.axis_index("s")
    iota16 = lax.broadcasted_iota(jnp.int32, (16,), 0)

    def do_flush():
        pltpu.async_copy(h1_hbm.at[srcb], hrows, sem).wait()

        def epack(g, carry3):
            fs_vec = fb[pl.ds(g * 16, 16)]
            for l in range(16):
                e = g * 16 + l
                fs = fs_vec[l]
                h0 = hrows[e, pl.ds(0, 16)]
                h1v = hrows[e, pl.ds(16, 16)]
                g0 = fs * h0
                g1 = fs * h1v
                gb[e, pl.ds(0, 16)] = g0
                gb[e, pl.ds(16, 16)] = g1
                db[e, pl.ds(0, 16)] = h0 - g0
                db[e, pl.ds(16, 16)] = h1v - g1
            return carry3

        lax.fori_loop(0, 16, epack, 0)
        pltpu.sync_copy(db, acc.at[ib], add=True)
        pltpu.sync_copy(gb, acc.at[it], add=True)

    def flush_full():
        def mv(i, carry):
            sl = pl.ds(i * 16, 16)
            srcb[sl] = csrc[sl]
            ib[sl] = crb[sl]
            it[sl] = crt[sl]
            fb[sl] = cf[sl]
            return carry

        lax.fori_loop(0, 16, mv, 0)
        do_flush()

    def flush_masked(off, cur):
        def mv(i, carry):
            sl = pl.ds(i * 16, 16)
            sl2 = pl.ds(off + i * 16, 16)
            valid = (iota16 + (off + i * 16)) < cur
            srcb[sl] = jnp.where(valid, csrc[sl2], 0)
            ib[sl] = jnp.where(valid, crb[sl2], DUMP2)
            it[sl] = jnp.where(valid, crt[sl2], DUMP2)
            fb[sl] = cf[sl2]
            return carry

        lax.fori_loop(0, 16, mv, 0)
        do_flush()

    def one_pass(p, carry0):
        r = p * 2 + c
        base = r * R2
        pltpu.sync_copy(z_hbm.at[pl.ds(s * R2TILE, R2TILE)],
                        acc.at[pl.ds(s * R2TILE, R2TILE)])
        plsc.subcore_barrier()

        def process(rec, cur):
            def group(g, cur2):
                li = (iota16 + g * 16) * 4
                sv = plsc.load_gather(rec, [li])
                dv = plsc.load_gather(rec, [li + 1])
                a = plsc.bitcast(plsc.load_gather(rec, [li + 2]), jnp.float32)
                u = jnp.minimum(jnp.maximum(a, 0.0), 1.0) * 4.0
                bi = u.astype(jnp.int32)
                f = u - bi.astype(jnp.float32)
                top = jnp.minimum(bi + 1, 4)
                loc = dv - base
                inr = (loc >= 0) & (loc < R2)
                r5 = loc * 5
                csl = pl.ds(cur2, 16)
                plsc.store_compressed(csrc.at[csl], sv, mask=inr)
                plsc.store_compressed(crb.at[csl], r5 + bi + 8, mask=inr)
                plsc.store_compressed(crt.at[csl], r5 + top + 8, mask=inr)
                plsc.store_compressed(cf.at[csl], f, mask=inr)
                cnt = plsc.all_reduce_population_count(inr)[0]
                return cur2 + cnt

            cur = lax.fori_loop(0, 16, group, cur)

            def spill(cv):
                flush_full()

                def sh(i, carry):
                    sl = pl.ds(i * 16, 16)
                    sl2 = pl.ds(256 + i * 16, 16)
                    csrc[sl] = csrc[sl2]
                    crb[sl] = crb[sl2]
                    crt[sl] = crt[sl2]
                    cf[sl] = cf[sl2]
                    return carry

                lax.fori_loop(0, 16, sh, 0)
                return cv - 256

            return lax.cond(cur >= 256, spill, lambda cv: cv, cur)

        ebase = s * (EP // 16)

        def rec_slice(b):
            return rec_hbm.at[pl.ds((ebase + b * 256) * 4, 1024)]

        pltpu.async_copy(rec_slice(0), recA, semA)

        def batch2(i, cur):
            pltpu.make_async_copy(rec_slice(2 * i), recA, semA).wait()
            pltpu.async_copy(rec_slice(2 * i + 1), recB, semB)
            cur = process(recA, cur)
            pltpu.make_async_copy(rec_slice(2 * i + 1), recB, semB).wait()
            pltpu.async_copy(rec_slice(2 * i + 2), recA, semA)
            return process(recB, cur)

        cur = lax.fori_loop(0, EP // 16 // 512, batch2, 0)
        pltpu.make_async_copy(rec_slice(EP // 16 // 256), recA, semA).wait()
        flush_masked(0, cur)
        flush_masked(256, cur)
        plsc.subcore_barrier()

        @pl.when(s == 0)
        def _():
            pltpu.sync_copy(acc.at[pl.ds(8, R2TILE - 8)],
                            out_hbm.at[pl.ds(r * (R2 * 5), R2TILE - 8)])

        @pl.when(s > 0)
        def _():
            pltpu.sync_copy(
                acc.at[pl.ds(s * R2TILE, R2TILE)],
                out_hbm.at[pl.ds(r * (R2 * 5) + s * R2TILE - 8, R2TILE)])

        plsc.subcore_barrier()
        return carry0

    lax.fori_loop(0, 9, one_pass, 0)


_l2_call = pl.kernel(
    _l2_body,
    out_type=[jax.ShapeDtypeStruct((OUTROWS, 32), jnp.float32)],
    mesh=_mesh,
    compiler_params=_sc_params,
    scratch_types=[
        pltpu.VMEM((1024,), jnp.int32),
        pltpu.VMEM((1024,), jnp.int32),
        pltpu.VMEM((256,), jnp.int32),
        pltpu.VMEM((256,), jnp.int32),
        pltpu.VMEM((256,), jnp.int32),
        pltpu.VMEM((256,), jnp.float32),
        pltpu.VMEM((256, 32), jnp.float32),
        pltpu.VMEM((256, 32), jnp.float32),
        pltpu.VMEM((256, 32), jnp.float32),
        pltpu.VMEM((512,), jnp.int32),
        pltpu.VMEM((512,), jnp.int32),
        pltpu.VMEM((512,), jnp.int32),
        pltpu.VMEM((512,), jnp.float32),
        pltpu.VMEM_SHARED((R2ROWS, 32), jnp.float32),
        pltpu.SemaphoreType.DMA,
        pltpu.SemaphoreType.DMA,
        pltpu.SemaphoreType.DMA,
    ],
)


def _tca_body(s1_ref, x_ref, w1f_ref, root1_ref, b1_ref, h1_ref, inv_ref):
    sm = s1_ref[0] + s1_ref[1]
    inv = 1.0 / jnp.maximum(sm[:, 5:6], 1.0)
    acc = sm @ w1f_ref[...]
    z = acc * inv + x_ref[...] * root1_ref[...] + b1_ref[...]
    h1_ref[...] = jnp.where(z > 0.0, z, jnp.exp(jnp.minimum(z, 0.0)) - 1.0)
    inv_ref[...] = jnp.broadcast_to(inv, (BLK, 8))


def _tca(s1r, x, w1f, root1, b1):
    return pl.pallas_call(
        _tca_body,
        grid=(N // BLK,),
        in_specs=[
            pl.BlockSpec((2, BLK, 8), lambda i: (0, i, 0)),
            pl.BlockSpec((BLK, 1), lambda i: (i, 0)),
            pl.BlockSpec((8, 32), lambda i: (0, 0)),
            pl.BlockSpec((1, 32), lambda i: (0, 0)),
            pl.BlockSpec((1, 32), lambda i: (0, 0)),
        ],
        out_specs=[
            pl.BlockSpec((BLK, 32), lambda i: (i, 0)),
            pl.BlockSpec((BLK, 8), lambda i: (i, 0)),
        ],
        out_shape=[
            jax.ShapeDtypeStruct((N, 32), jnp.float32),
            jax.ShapeDtypeStruct((N, 8), jnp.float32),
        ],
    )(s1r, x, w1f, root1, b1)


def _tcb_body(s2_ref, h1_ref, inv_ref, w2f_ref, root2_ref, b2_ref,
              l1w_ref, l1b_ref, l2w_ref, l2b_ref, out_ref):
    inv = inv_ref[:, 0:1]
    z = s2_ref[...] @ w2f_ref[...] * inv + h1_ref[...] @ root2_ref[...] + b2_ref[...]
    h2 = jnp.where(z > 0.0, z, jnp.exp(jnp.minimum(z, 0.0)) - 1.0)
    a = h2 @ l1w_ref[...] + l1b_ref[...]
    t = jnp.where(a > 0.0, a, jnp.exp(jnp.minimum(a, 0.0)) - 1.0)
    o = t @ l2w_ref[...] + l2b_ref[...]
    mask = lax.broadcasted_iota(jnp.int32, o.shape, 1) < 10
    o = jnp.where(mask, o, -1e30)
    m = jnp.max(o, axis=1, keepdims=True)
    lse = jnp.log(jnp.sum(jnp.exp(o - m), axis=1, keepdims=True)) + m
    out_ref[...] = o - lse


def _tcb(s2, h1, inv8, w2f, root2, b2, l1w, l1b, l2wp, l2bp):
    return pl.pallas_call(
        _tcb_body,
        grid=(N // BLK,),
        in_specs=[
            pl.BlockSpec((BLK, 160), lambda i: (i, 0)),
            pl.BlockSpec((BLK, 32), lambda i: (i, 0)),
            pl.BlockSpec((BLK, 8), lambda i: (i, 0)),
            pl.BlockSpec((160, 64), lambda i: (0, 0)),
            pl.BlockSpec((32, 64), lambda i: (0, 0)),
            pl.BlockSpec((1, 64), lambda i: (0, 0)),
            pl.BlockSpec((64, 128), lambda i: (0, 0)),
            pl.BlockSpec((1, 128), lambda i: (0, 0)),
            pl.BlockSpec((128, 128), lambda i: (0, 0)),
            pl.BlockSpec((1, 128), lambda i: (0, 0)),
        ],
        out_specs=pl.BlockSpec((BLK, 128), lambda i: (i, 0)),
        out_shape=jax.ShapeDtypeStruct((N, 128), jnp.float32),
    )(s2, h1, inv8, w2f, root2, b2, l1w, l1b, l2wp, l2bp)


def kernel(x, edge_index, edge_attr, W1, root1, b1, W2, root2, b2, l1w, l1b, l2w, l2b):
    src = edge_index[0].astype(jnp.int32)
    dst = edge_index[1].astype(jnp.int32)
    attr = edge_attr[:, 0]
    pad = EP - E
    src_p = jnp.concatenate([src, jnp.zeros((pad,), jnp.int32)])
    dst_p = jnp.concatenate([dst, jnp.full((pad,), N, jnp.int32)])
    attr_p = jnp.concatenate([attr, jnp.zeros((pad,), jnp.float32)])
    attr_bits = lax.bitcast_convert_type(attr_p, jnp.int32)
    recs = jnp.stack([src_p, dst_p, attr_bits, jnp.zeros_like(src_p)],
                     axis=1).reshape(-1)
    recs = jnp.concatenate([recs, jnp.zeros((1024,), jnp.int32)])

    x_flat = x[:, 0]
    z1 = jnp.zeros((L1ACC,), jnp.float32)
    (s1_pair,) = _l1_call(x_flat, recs, z1)
    s1r = s1_pair.reshape(2, L1ACC)[:, : N * 8].reshape(2, N, 8)

    w1f = jnp.pad(W1[:, 0, :], ((0, 3), (0, 0)))
    h1, inv8 = _tca(s1r, x, w1f, root1, b1.reshape(1, 32))

    z2 = jnp.zeros((R2ROWS, 32), jnp.float32)
    (s2_flat,) = _l2_call(h1, recs, z2)
    s2 = s2_flat[: N * 5].reshape(N, 160)

    w2f = W2.reshape(160, 64)
    l2wp = jnp.pad(l2w, ((0, 0), (0, 118)))
    l2bp = jnp.pad(l2b, (0, 118)).reshape(1, 128)
    out = _tcb(s2, h1, inv8, w2f, root2, b2.reshape(1, 64),
               l1w, l1b.reshape(1, 128), l2wp, l2bp)
    return out[:, :10]
